# Initial kernel scaffold; baseline (speedup 1.0000x reference)
import functools

import jax
import jax.numpy as jnp
from jax import lax
from jax.experimental import pallas as pl
from jax.experimental.pallas import tpu as pltpu
from jax.experimental.pallas import tpu_sc as plsc

N_NODES = 10000
N_EDGES = 160000
D_HID = 16
NC = 2
NS = 16
E_PER_TILE = N_EDGES // (NC * NS)
ROWS_PER_TILE = N_NODES // NS


def _make_sc_spmm():
    mesh = plsc.VectorSubcoreMesh(core_axis_name="c", subcore_axis_name="s")

    @functools.partial(
        pl.kernel,
        mesh=mesh,
        out_type=jax.ShapeDtypeStruct((NC, N_NODES, D_HID), jnp.float32),
        scratch_types=[
            pltpu.VMEM((E_PER_TILE,), jnp.int32),
            pltpu.VMEM((E_PER_TILE,), jnp.int32),
            pltpu.VMEM((E_PER_TILE,), jnp.float32),
            pltpu.VMEM((E_PER_TILE, D_HID), jnp.float32),
            pltpu.VMEM((ROWS_PER_TILE, D_HID), jnp.float32),
            pltpu.VMEM_SHARED((N_NODES, D_HID), jnp.float32),
            pltpu.SemaphoreType.DMA,
        ],
    )
    def spmm(rows_hbm, cols_hbm, vals_hbm, h_hbm, out_hbm,
             rows_v, cols_v, vals_v, gath_v, zbuf_v, acc_sh, sem):
        c = lax.axis_index("c")
        s = lax.axis_index("s")
        base = (c * NS + s) * E_PER_TILE

        def zero_body(j, _):
            zbuf_v[j, :] = jnp.zeros((D_HID,), jnp.float32)
            return 0
        lax.fori_loop(0, ROWS_PER_TILE, zero_body, 0)
        pltpu.sync_copy(zbuf_v,
                        acc_sh.at[pl.ds(s * ROWS_PER_TILE, ROWS_PER_TILE)])

        pltpu.sync_copy(rows_hbm.at[pl.ds(base, E_PER_TILE)], rows_v)
        pltpu.sync_copy(cols_hbm.at[pl.ds(base, E_PER_TILE)], cols_v)
        pltpu.sync_copy(vals_hbm.at[pl.ds(base, E_PER_TILE)], vals_v)

        pltpu.async_copy(h_hbm.at[cols_v], gath_v, sem).wait()

        def scale_body(e, _):
            splat = plsc.load_gather(
                vals_v, [jnp.full((D_HID,), e, jnp.int32)])
            gath_v[e, :] = gath_v[e, :] * splat
            return 0
        lax.fori_loop(0, E_PER_TILE, scale_body, 0)

        plsc.subcore_barrier()

        pltpu.sync_copy(gath_v, acc_sh.at[rows_v], add=True)

        plsc.subcore_barrier()

        pltpu.sync_copy(acc_sh.at[pl.ds(s * ROWS_PER_TILE, ROWS_PER_TILE)],
                        out_hbm.at[c, pl.ds(s * ROWS_PER_TILE, ROWS_PER_TILE)])

    return spmm


_sc_spmm = _make_sc_spmm()


def _tc_matmul1(feature, w1):
    m, k = feature.shape
    n = w1.shape[1]
    bm = 2000

    def body(x_ref, w_ref, o_ref):
        o_ref[:] = jnp.dot(x_ref[:], w_ref[:],
                           preferred_element_type=jnp.float32)

    return pl.pallas_call(
        body,
        grid=(m // bm,),
        in_specs=[pl.BlockSpec((bm, k), lambda i: (i, 0)),
                  pl.BlockSpec((k, n), lambda i: (0, 0))],
        out_specs=pl.BlockSpec((bm, n), lambda i: (i, 0)),
        out_shape=jax.ShapeDtypeStruct((m, n), jnp.float32),
    )(feature, w1)


def _tc_bias_relu(partials, b1):
    _, m, n = partials.shape
    bm = 2000

    def body(p_ref, b_ref, o_ref):
        o_ref[:] = jnp.maximum(p_ref[0] + p_ref[1] + b_ref[:], 0.0)

    return pl.pallas_call(
        body,
        grid=(m // bm,),
        in_specs=[pl.BlockSpec((NC, bm, n), lambda i: (0, i, 0)),
                  pl.BlockSpec((1, n), lambda i: (0, 0))],
        out_specs=pl.BlockSpec((bm, n), lambda i: (i, 0)),
        out_shape=jax.ShapeDtypeStruct((m, n), jnp.float32),
    )(partials, b1)


def _tc_final(partials, w2, b2):
    _, m, k = partials.shape
    n = w2.shape[1]
    bm = 2000

    def body(p_ref, w_ref, b_ref, o_ref):
        g = p_ref[0] + p_ref[1]
        o_ref[:] = jnp.dot(g, w_ref[:],
                           preferred_element_type=jnp.float32) + b_ref[:]

    return pl.pallas_call(
        body,
        grid=(m // bm,),
        in_specs=[pl.BlockSpec((NC, bm, k), lambda i: (0, i, 0)),
                  pl.BlockSpec((k, n), lambda i: (0, 0)),
                  pl.BlockSpec((1, n), lambda i: (0, 0))],
        out_specs=pl.BlockSpec((bm, n), lambda i: (i, 0)),
        out_shape=jax.ShapeDtypeStruct((m, n), jnp.float32),
    )(partials, w2, b2)


def kernel(adjacency_edge_index, adjacency_values, feature, W1, b1, W2, b2):
    rows = adjacency_edge_index[0]
    cols = adjacency_edge_index[1]

    h1 = _tc_matmul1(feature, W1)
    p1 = _sc_spmm(rows, cols, adjacency_values, h1)
    h = _tc_bias_relu(p1, b1.reshape(1, -1))
    p2 = _sc_spmm(rows, cols, adjacency_values, h)
    return _tc_final(p2, W2, b2.reshape(1, -1))


# speedup vs baseline: 12.2407x; 12.2407x over previous
import functools

import jax
import jax.numpy as jnp
from jax import lax
from jax.experimental import pallas as pl
from jax.experimental.pallas import tpu as pltpu
from jax.experimental.pallas import tpu_sc as plsc

N_NODES = 10000
N_PAD = 10112
N_EDGES = 160000
E_PAD = 160256
D_HID = 16
NC = 2
NS = 16
E_PER_TILE = E_PAD // (NC * NS)
ROWS_PER_TILE = N_PAD // NS


def _make_sc_spmm():
    mesh = plsc.VectorSubcoreMesh(core_axis_name="c", subcore_axis_name="s")

    @functools.partial(
        pl.kernel,
        mesh=mesh,
        compiler_params=pltpu.CompilerParams(use_tc_tiling_on_sc=False),
        out_type=jax.ShapeDtypeStruct((NC, N_PAD, D_HID), jnp.float32),
        scratch_types=[
            pltpu.VMEM((E_PER_TILE,), jnp.int32),
            pltpu.VMEM((E_PER_TILE,), jnp.int32),
            pltpu.VMEM((E_PER_TILE,), jnp.float32),
            pltpu.VMEM((E_PER_TILE, D_HID), jnp.float32),
            pltpu.VMEM((ROWS_PER_TILE, D_HID), jnp.float32),
            pltpu.VMEM_SHARED((N_PAD, D_HID), jnp.float32),
            pltpu.SemaphoreType.DMA,
        ],
    )
    def spmm(rows_hbm, cols_hbm, vals_hbm, h_hbm, out_hbm,
             rows_v, cols_v, vals_v, gath_v, zbuf_v, acc_sh, sem):
        c = lax.axis_index("c")
        s = lax.axis_index("s")
        base = (c * NS + s) * E_PER_TILE

        def zero_body(j, _):
            zbuf_v[j, :] = jnp.zeros((D_HID,), jnp.float32)
            return 0
        lax.fori_loop(0, ROWS_PER_TILE, zero_body, 0)
        pltpu.sync_copy(zbuf_v,
                        acc_sh.at[pl.ds(s * ROWS_PER_TILE, ROWS_PER_TILE)])

        pltpu.sync_copy(rows_hbm.at[pl.ds(base, E_PER_TILE)], rows_v)
        pltpu.sync_copy(cols_hbm.at[pl.ds(base, E_PER_TILE)], cols_v)
        pltpu.sync_copy(vals_hbm.at[pl.ds(base, E_PER_TILE)], vals_v)

        pltpu.async_copy(h_hbm.at[cols_v], gath_v, sem).wait()

        def scale_body(g, _):
            vals = vals_v[pl.ds(g * 16, 16)]
            for j in range(16):
                e = g * 16 + j
                splat = lax.gather(
                    vals, jnp.full((16, 1), j, jnp.int32),
                    dimension_numbers=lax.GatherDimensionNumbers(
                        offset_dims=(), collapsed_slice_dims=(0,),
                        start_index_map=(0,)),
                    slice_sizes=(1,),
                    mode=lax.GatherScatterMode.PROMISE_IN_BOUNDS)
                gath_v[e, :] = gath_v[e, :] * splat
            return 0
        lax.fori_loop(0, E_PER_TILE // 16, scale_body, 0)

        plsc.subcore_barrier()

        pltpu.sync_copy(gath_v, acc_sh.at[rows_v], add=True)

        plsc.subcore_barrier()

        pltpu.sync_copy(acc_sh.at[pl.ds(s * ROWS_PER_TILE, ROWS_PER_TILE)],
                        out_hbm.at[c, pl.ds(s * ROWS_PER_TILE, ROWS_PER_TILE)])

    return spmm


_sc_spmm = _make_sc_spmm()


def _tc_matmul1(feature, w1):
    m, k = feature.shape
    n = w1.shape[1]
    bm = 632

    def body(x_ref, w_ref, o_ref):
        o_ref[:] = jnp.dot(x_ref[:], w_ref[:],
                           preferred_element_type=jnp.float32)

    return pl.pallas_call(
        body,
        grid=(m // bm,),
        in_specs=[pl.BlockSpec((bm, k), lambda i: (i, 0)),
                  pl.BlockSpec((k, n), lambda i: (0, 0))],
        out_specs=pl.BlockSpec((bm, n), lambda i: (i, 0)),
        out_shape=jax.ShapeDtypeStruct((m, n), jnp.float32),
    )(feature, w1)


def _tc_bias_relu(partials, b1):
    _, m, n = partials.shape
    bm = 632

    def body(p_ref, b_ref, o_ref):
        o_ref[:] = jnp.maximum(p_ref[0] + p_ref[1] + b_ref[:], 0.0)

    return pl.pallas_call(
        body,
        grid=(m // bm,),
        in_specs=[pl.BlockSpec((NC, bm, n), lambda i: (0, i, 0)),
                  pl.BlockSpec((1, n), lambda i: (0, 0))],
        out_specs=pl.BlockSpec((bm, n), lambda i: (i, 0)),
        out_shape=jax.ShapeDtypeStruct((m, n), jnp.float32),
    )(partials, b1)


def _tc_final(partials, w2, b2):
    _, m, k = partials.shape
    n = w2.shape[1]
    bm = 632

    def body(p_ref, w_ref, b_ref, o_ref):
        g = p_ref[0] + p_ref[1]
        o_ref[:] = jnp.dot(g, w_ref[:],
                           preferred_element_type=jnp.float32) + b_ref[:]

    return pl.pallas_call(
        body,
        grid=(m // bm,),
        in_specs=[pl.BlockSpec((NC, bm, k), lambda i: (0, i, 0)),
                  pl.BlockSpec((k, n), lambda i: (0, 0)),
                  pl.BlockSpec((1, n), lambda i: (0, 0))],
        out_specs=pl.BlockSpec((bm, n), lambda i: (i, 0)),
        out_shape=jax.ShapeDtypeStruct((m, n), jnp.float32),
    )(partials, w2, b2)


def kernel(adjacency_edge_index, adjacency_values, feature, W1, b1, W2, b2):
    epad = E_PAD - N_EDGES
    rows = jnp.pad(adjacency_edge_index[0], (0, epad))
    cols = jnp.pad(adjacency_edge_index[1], (0, epad))
    vals = jnp.pad(adjacency_values, (0, epad))
    feature_p = jnp.pad(feature, ((0, N_PAD - N_NODES), (0, 0)))

    h1 = _tc_matmul1(feature_p, W1)
    p1 = _sc_spmm(rows, cols, vals, h1)
    h = _tc_bias_relu(p1, b1.reshape(1, -1))
    p2 = _sc_spmm(rows, cols, vals, h)
    return _tc_final(p2, W2, b2.reshape(1, -1))[:N_NODES]


# speedup vs baseline: 13.8785x; 1.1338x over previous
import functools

import jax
import jax.numpy as jnp
from jax import lax
from jax.experimental import pallas as pl
from jax.experimental.pallas import tpu as pltpu
from jax.experimental.pallas import tpu_sc as plsc

N_NODES = 10000
N_PAD = 10112
N_EDGES = 160000
E_PAD = 160256
D_HID = 16
NC = 2
NS = 16
E_PER_TILE = E_PAD // (NC * NS)
ROWS_PER_TILE = N_PAD // NS


def _splat_lane(vec, j):
    return lax.gather(
        vec, jnp.full((D_HID, 1), j, jnp.int32),
        dimension_numbers=lax.GatherDimensionNumbers(
            offset_dims=(), collapsed_slice_dims=(0,), start_index_map=(0,)),
        slice_sizes=(1,),
        mode=lax.GatherScatterMode.PROMISE_IN_BOUNDS)


def _make_sc_spmm(fused):
    mesh = plsc.VectorSubcoreMesh(core_axis_name="c", subcore_axis_name="s")

    out_slab = jax.ShapeDtypeStruct((N_PAD, D_HID), jnp.float32)
    scratch = [
        pltpu.VMEM((E_PER_TILE,), jnp.int32),
        pltpu.VMEM((E_PER_TILE,), jnp.int32),
        pltpu.VMEM((E_PER_TILE,), jnp.float32),
        pltpu.VMEM((E_PER_TILE, D_HID), jnp.float32),
        pltpu.VMEM((ROWS_PER_TILE, D_HID), jnp.float32),
        pltpu.VMEM((D_HID,), jnp.float32),
        pltpu.VMEM_SHARED((N_PAD, D_HID), jnp.float32),
        pltpu.SemaphoreType.DMA,
    ]

    def body(refs):
        if fused:
            (rows_hbm, cols_hbm, vals_hbm, h0_hbm, h1_hbm, b_hbm,
             out0_hbm, out1_hbm,
             rows_v, cols_v, vals_v, gath_v, zbuf_v, b_v, acc_sh, sem) = refs
        else:
            (rows_hbm, cols_hbm, vals_hbm, h0_hbm,
             out0_hbm, out1_hbm,
             rows_v, cols_v, vals_v, gath_v, zbuf_v, b_v, acc_sh, sem) = refs
        c = lax.axis_index("c")
        s = lax.axis_index("s")
        base = (c * NS + s) * E_PER_TILE

        cp_rows = pltpu.async_copy(
            rows_hbm.at[pl.ds(base, E_PER_TILE)], rows_v, sem)
        cp_cols = pltpu.async_copy(
            cols_hbm.at[pl.ds(base, E_PER_TILE)], cols_v, sem)
        cp_vals = pltpu.async_copy(
            vals_hbm.at[pl.ds(base, E_PER_TILE)], vals_v, sem)

        def zero_body(j, _):
            zbuf_v[j, :] = jnp.zeros((D_HID,), jnp.float32)
            return 0
        lax.fori_loop(0, ROWS_PER_TILE, zero_body, 0)
        pltpu.sync_copy(zbuf_v,
                        acc_sh.at[pl.ds(s * ROWS_PER_TILE, ROWS_PER_TILE)])

        cp_rows.wait()
        cp_cols.wait()
        cp_vals.wait()

        pltpu.async_copy(h0_hbm.at[cols_v], gath_v, sem).wait()
        if fused:
            pltpu.async_copy(h1_hbm.at[cols_v], gath_v, sem,
                             add=True).wait()
            pltpu.sync_copy(b_hbm, b_v)
            bias = b_v[:]

        def scale_body(g, _):
            vals16 = vals_v[pl.ds(g * 16, 16)]
            for j in range(16):
                e = g * 16 + j
                row = gath_v[e, :]
                if fused:
                    row = jnp.maximum(row + bias, 0.0)
                gath_v[e, :] = row * _splat_lane(vals16, j)
            return 0
        lax.fori_loop(0, E_PER_TILE // 16, scale_body, 0)

        plsc.subcore_barrier()

        pltpu.sync_copy(gath_v, acc_sh.at[rows_v], add=True)

        plsc.subcore_barrier()

        acc_slice = acc_sh.at[pl.ds(s * ROWS_PER_TILE, ROWS_PER_TILE)]

        @pl.when(c == 0)
        def _():
            pltpu.sync_copy(
                acc_slice,
                out0_hbm.at[pl.ds(s * ROWS_PER_TILE, ROWS_PER_TILE)])

        @pl.when(c == 1)
        def _():
            pltpu.sync_copy(
                acc_slice,
                out1_hbm.at[pl.ds(s * ROWS_PER_TILE, ROWS_PER_TILE)])

    def entry(*refs):
        body(refs)

    return pl.kernel(
        entry,
        mesh=mesh,
        compiler_params=pltpu.CompilerParams(use_tc_tiling_on_sc=False),
        out_type=(out_slab, out_slab),
        scratch_types=scratch,
    )


_sc_spmm = _make_sc_spmm(fused=False)
_sc_spmm_fused = _make_sc_spmm(fused=True)


def _tc_matmul1(feature, w1):
    m, k = feature.shape
    n = w1.shape[1]
    bm = 632

    def body(x_ref, w_ref, o_ref):
        o_ref[:] = jnp.dot(x_ref[:], w_ref[:],
                           preferred_element_type=jnp.float32)

    return pl.pallas_call(
        body,
        grid=(m // bm,),
        in_specs=[pl.BlockSpec((bm, k), lambda i: (i, 0)),
                  pl.BlockSpec((k, n), lambda i: (0, 0))],
        out_specs=pl.BlockSpec((bm, n), lambda i: (i, 0)),
        out_shape=jax.ShapeDtypeStruct((m, n), jnp.float32),
    )(feature, w1)


def _tc_final(q0, q1, w2, b2):
    m, k = q0.shape
    n = w2.shape[1]
    bm = 632

    def body(p_ref, q_ref, w_ref, b_ref, o_ref):
        g = p_ref[:] + q_ref[:]
        o_ref[:] = jnp.dot(g, w_ref[:],
                           preferred_element_type=jnp.float32) + b_ref[:]

    return pl.pallas_call(
        body,
        grid=(m // bm,),
        in_specs=[pl.BlockSpec((bm, k), lambda i: (i, 0)),
                  pl.BlockSpec((bm, k), lambda i: (i, 0)),
                  pl.BlockSpec((k, n), lambda i: (0, 0)),
                  pl.BlockSpec((1, n), lambda i: (0, 0))],
        out_specs=pl.BlockSpec((bm, n), lambda i: (i, 0)),
        out_shape=jax.ShapeDtypeStruct((m, n), jnp.float32),
    )(q0, q1, w2, b2)


def kernel(adjacency_edge_index, adjacency_values, feature, W1, b1, W2, b2):
    epad = E_PAD - N_EDGES
    rows = jnp.pad(adjacency_edge_index[0], (0, epad))
    cols = jnp.pad(adjacency_edge_index[1], (0, epad))
    vals = jnp.pad(adjacency_values, (0, epad))
    feature_p = jnp.pad(feature, ((0, N_PAD - N_NODES), (0, 0)))

    h1 = _tc_matmul1(feature_p, W1)
    p0, p1 = _sc_spmm(rows, cols, vals, h1)
    q0, q1 = _sc_spmm_fused(rows, cols, vals, p0, p1, b1)
    return _tc_final(q0, q1, W2, b2.reshape(1, -1))[:N_NODES]
